# baseline (speedup 1.0000x reference)
import jax
import jax.numpy as jnp
from jax.experimental import pallas as pl

_B, _C, _L = 512, 64, 2048
_DS = 32
_LS = _L // _DS
_BB = 8
_R = _BB * _LS


def _pre_block_body(x_ref, wmat_ref, convb_ref, cb_ref, cbh_ref, w1_ref,
                    b1_ref, w2_ref, b2_ref, out_ref):
    wmatT = wmat_ref[...]
    conv_b = convb_ref[...]
    cb = cb_ref[...]
    cbh = cbh_ref[...]
    w1 = w1_ref[...]
    b1 = b1_ref[...]
    w2 = w2_ref[...]
    b2 = b2_ref[...]
    cb_sq = jnp.sum(cb * cb, axis=1, keepdims=True).T

    xm = x_ref[...].reshape(_R, _C * _DS)
    y = jnp.dot(xm, wmatT, preferred_element_type=jnp.float32) + conv_b

    x_de = jnp.transpose(y.reshape(_BB, _LS, _C), (0, 2, 1)).reshape(_R, _LS)

    dotc = jax.lax.dot_general(
        x_de.astype(jnp.bfloat16), cbh, (((1,), (1,)), ((), ())),
        preferred_element_type=jnp.float32)
    x_sq = jnp.sum(x_de * x_de, axis=1, keepdims=True)
    d2 = jnp.maximum(x_sq + cb_sq - 2.0 * dotc, 0.0)
    idx = jnp.argmin(d2, axis=1, keepdims=True)

    iota = jax.lax.broadcasted_iota(jnp.int32, (_R, _LS), 1)
    onehot = (iota == idx).astype(jnp.float32)
    q = jnp.dot(onehot, cb, preferred_element_type=jnp.float32)

    t = x_de - q
    tp = jnp.transpose(t.reshape(_BB, _C, _LS), (0, 2, 1)).reshape(_R, _C)
    h = jax.lax.dot_general(
        tp.astype(jnp.bfloat16), w1, (((1,), (1,)), ((), ())),
        preferred_element_type=jnp.float32) + b1
    h = jnp.maximum(h, 0.0)
    mp = jax.lax.dot_general(
        h.astype(jnp.bfloat16), w2, (((1,), (1,)), ((), ())),
        preferred_element_type=jnp.float32) + b2
    mpT = jnp.transpose(mp.reshape(_BB, _LS, _C), (0, 2, 1)).reshape(_R, _LS)
    out_ref[...] = (mpT + q).reshape(_BB, _C, _LS)


def kernel(x, conv_w, conv_b, codebook, W1, b1, W2, b2):
    xprep = jnp.transpose(
        x.astype(jnp.bfloat16).reshape(_B, _C, _LS, _DS),
        (0, 2, 1, 3)).reshape(_B, _LS, _C * _DS)
    wmatT = conv_w.reshape(_C, _C * _DS).T.astype(jnp.bfloat16)
    convb2 = conv_b.reshape(1, _C)
    cbh = codebook.astype(jnp.bfloat16)
    w1h = W1.astype(jnp.bfloat16)
    w2h = W2.astype(jnp.bfloat16)
    b1r = b1.reshape(1, _LS)
    b2r = b2.reshape(1, _LS)

    grid = (_B // _BB,)
    full = lambda i: (0, 0)
    out = pl.pallas_call(
        _pre_block_body,
        grid=grid,
        in_specs=[
            pl.BlockSpec((_BB, _LS, _C * _DS), lambda i: (i, 0, 0)),
            pl.BlockSpec((_C * _DS, _C), full),
            pl.BlockSpec((1, _C), full),
            pl.BlockSpec((_LS, _C), full),
            pl.BlockSpec((_LS, _C), full),
            pl.BlockSpec((_LS, _LS), full),
            pl.BlockSpec((1, _LS), full),
            pl.BlockSpec((_LS, _LS), full),
            pl.BlockSpec((1, _LS), full),
        ],
        out_specs=pl.BlockSpec((_BB, _C, _LS), lambda i: (i, 0, 0)),
        out_shape=jax.ShapeDtypeStruct((_B, _C, _LS), jnp.float32),
    )(xprep, wmatT, convb2, codebook, cbh, w1h, b1r, w2h, b2r)
    return out


# speedup vs baseline: 2.7995x; 2.7995x over previous
import jax
import jax.numpy as jnp
from jax.experimental import pallas as pl
from jax.experimental.pallas import tpu as pltpu

_B, _C, _L = 512, 64, 2048
_DS = 32
_LS = _L // _DS
_BB = 8
_R = _BB * _LS


def _pre_block_body(x_ref, wmat_ref, convb_ref, cb_ref, cbh_ref, w1_ref,
                    b1_ref, w2_ref, b2_ref, out_ref):
    wmatT = wmat_ref[...]
    conv_b = convb_ref[...]
    cb = cb_ref[...]
    cbh = cbh_ref[...]
    w1 = w1_ref[...]
    b1 = b1_ref[...]
    w2 = w2_ref[...]
    b2 = b2_ref[...]
    cb_sq = jnp.sum(cb * cb, axis=1, keepdims=True).T

    xall = x_ref[...].astype(jnp.bfloat16).reshape(_BB, _C, _LS, _DS)
    xm = jnp.concatenate([xall[:, c].reshape(_R, _DS) for c in range(_C)],
                         axis=1)
    y = jnp.dot(xm, wmatT, preferred_element_type=jnp.float32) + conv_b

    x_de = jnp.transpose(y.reshape(_BB, _LS, _C), (0, 2, 1)).reshape(_R, _LS)

    dotc = jax.lax.dot_general(
        x_de.astype(jnp.bfloat16), cbh, (((1,), (1,)), ((), ())),
        preferred_element_type=jnp.float32)
    x_sq = jnp.sum(x_de * x_de, axis=1, keepdims=True)
    d2 = jnp.maximum(x_sq + cb_sq - 2.0 * dotc, 0.0)
    idx = jnp.argmin(d2, axis=1, keepdims=True)

    iota = jax.lax.broadcasted_iota(jnp.int32, (_R, _LS), 1)
    onehot = (iota == idx).astype(jnp.float32)
    q = jnp.dot(onehot, cb, preferred_element_type=jnp.float32)

    t = x_de - q
    tp = jnp.transpose(t.reshape(_BB, _C, _LS), (0, 2, 1)).reshape(_R, _C)
    h = jax.lax.dot_general(
        tp.astype(jnp.bfloat16), w1, (((1,), (1,)), ((), ())),
        preferred_element_type=jnp.float32) + b1
    h = jnp.maximum(h, 0.0)
    mp = jax.lax.dot_general(
        h.astype(jnp.bfloat16), w2, (((1,), (1,)), ((), ())),
        preferred_element_type=jnp.float32) + b2
    mpT = jnp.transpose(mp.reshape(_BB, _LS, _C), (0, 2, 1)).reshape(_R, _LS)
    out_ref[...] = (mpT + q).reshape(_BB, _C, _LS)


def kernel(x, conv_w, conv_b, codebook, W1, b1, W2, b2):
    wmatT = conv_w.reshape(_C, _C * _DS).T.astype(jnp.bfloat16)
    convb2 = conv_b.reshape(1, _C)
    cbh = codebook.astype(jnp.bfloat16)
    w1h = W1.astype(jnp.bfloat16)
    w2h = W2.astype(jnp.bfloat16)
    b1r = b1.reshape(1, _LS)
    b2r = b2.reshape(1, _LS)

    grid = (_B // _BB,)
    full = lambda i: (0, 0)
    out = pl.pallas_call(
        _pre_block_body,
        grid=grid,
        in_specs=[
            pl.BlockSpec((_BB, _C, _L), lambda i: (i, 0, 0)),
            pl.BlockSpec((_C * _DS, _C), full),
            pl.BlockSpec((1, _C), full),
            pl.BlockSpec((_LS, _C), full),
            pl.BlockSpec((_LS, _C), full),
            pl.BlockSpec((_LS, _LS), full),
            pl.BlockSpec((1, _LS), full),
            pl.BlockSpec((_LS, _LS), full),
            pl.BlockSpec((1, _LS), full),
        ],
        out_specs=pl.BlockSpec((_BB, _C, _LS), lambda i: (i, 0, 0)),
        out_shape=jax.ShapeDtypeStruct((_B, _C, _LS), jnp.float32),
        compiler_params=pltpu.CompilerParams(
            dimension_semantics=("parallel",)),
    )(x, wmatT, convb2, codebook, cbh, w1h, b1r, w2h, b2r)
    return out
